# baseline (speedup 1.0000x reference)
import jax
import jax.numpy as jnp
from jax import lax
from jax.experimental import pallas as pl
from jax.experimental.pallas import tpu as pltpu
from jax.experimental.pallas import tpu_sc as plsc

B = 64
H = 224
W = 224
CIN = 5
COUT = 3

TROWS_PER_PLANE = 2 * H
TROWS_PER_IMG = COUT * TROWS_PER_PLANE
TBL_ROWS = B * TROWS_PER_IMG
RIGHT_OFF = 28576

NC = 2
NS = 16
NW = NC * NS
BH = B // 2
OCH = 16
NCHUNK = H // OCH


def _tc_body(x_ref, tbl_ref, idx_ref):
    v = x_ref[0]
    for c in range(COUT):
        p = v[c]
        s = (p[0:222, 0:222] + p[0:222, 2:224]
             + p[2:224, 0:222] + p[2:224, 2:224]) * 0.25
        r = c * TROWS_PER_PLANE
        tbl_ref[r:r + TROWS_PER_PLANE, :] = jnp.zeros(
            (TROWS_PER_PLANE, 128), jnp.float32)
        tbl_ref[r + 1:r + 223, 1:128] = s[:, 0:127]
        tbl_ref[r + H + 1:r + H + 223, 0:127] = s[:, 95:222]

    ii = lax.broadcasted_iota(jnp.int32, (H, W), 0).astype(jnp.float32)
    jj = lax.broadcasted_iota(jnp.int32, (H, W), 1).astype(jnp.float32)
    yi = jnp.clip(jnp.mod(ii + v[4], 224.0).astype(jnp.int32), 0, 223)
    xi = jnp.clip(jnp.mod(jj + v[3], 224.0).astype(jnp.int32), 0, 223)
    idx_ref[0] = yi * 128 + xi + jnp.where(xi >= 128, RIGHT_OFF, 0)


def _tc_stage(x_p, base):
    return pl.pallas_call(
        _tc_body,
        grid=(BH,),
        in_specs=[pl.BlockSpec((1, CIN, H, W),
                               lambda b: (b + base, 0, 0, 0))],
        out_specs=[
            pl.BlockSpec((TROWS_PER_IMG, 128), lambda b: (b, 0)),
            pl.BlockSpec((1, H, W), lambda b: (b, 0, 0)),
        ],
        out_shape=[
            jax.ShapeDtypeStruct((BH * TROWS_PER_IMG, 128), jnp.float32),
            jax.ShapeDtypeStruct((BH, H, W), jnp.int32),
        ],
    )(x_p)


def _sc_body(tbl_hbm, idx_hbm, out_hbm,
             tl0, tl1, ib0, ib1, ob0, ob1,
             st0, st1, si0, si1, so0, so1):
    wid = lax.axis_index("s") * NC + lax.axis_index("c")
    b = wid
    tbufs, ibufs, obufs = (tl0, tl1), (ib0, ib1), (ob0, ob1)
    stems, siems, soems = (st0, st1), (si0, si1), (so0, so1)

    def tbl_src(c):
        r0 = (b * TROWS_PER_IMG + c * TROWS_PER_PLANE) * 128
        return tbl_hbm.at[pl.ds(r0, TROWS_PER_PLANE * 128)]

    def idx_src(g):
        return idx_hbm.at[b, pl.ds(g * OCH, OCH)]

    pltpu.async_copy(tbl_src(0), tbufs[0], stems[0])
    pltpu.async_copy(idx_src(0), ibufs[0], siems[0])
    pend_out = [None, None]

    for c in range(COUT):
        tb = tbufs[c % 2]
        pltpu.make_async_copy(tbl_src(c), tb, stems[c % 2]).wait()
        if c + 1 < COUT:
            pltpu.async_copy(tbl_src(c + 1), tbufs[(c + 1) % 2],
                             stems[(c + 1) % 2])
        for gg in range(NCHUNK):
            q = c * NCHUNK + gg
            p = q % 2
            ib = ibufs[p]
            pltpu.make_async_copy(idx_src(gg), ib, siems[p]).wait()
            if q + 1 < COUT * NCHUNK:
                pltpu.async_copy(idx_src((gg + 1) % NCHUNK),
                                 ibufs[1 - p], siems[1 - p])
            if pend_out[p] is not None:
                ob_, od_ = pend_out[p]
                pltpu.make_async_copy(ob_, od_, soems[p]).wait()
            ob = obufs[p]

            def fill(rr, carry, _ib=ib, _ob=ob, _tb=tb):
                for t in range(W // 16):
                    pv = _ib[rr, pl.ds(t * 16, 16)]
                    _ob[rr, pl.ds(t * 16, 16)] = plsc.load_gather(_tb, [pv])
                return carry

            lax.fori_loop(0, OCH, fill, 0)
            dst = out_hbm.at[b, c, pl.ds(gg * OCH, OCH)]
            pltpu.async_copy(ob, dst, soems[p])
            pend_out[p] = (ob, dst)
    for p in range(2):
        if pend_out[p] is not None:
            ob_, od_ = pend_out[p]
            pltpu.make_async_copy(ob_, od_, soems[p]).wait()


def _sc_stage(tbl, idx):
    mesh = plsc.VectorSubcoreMesh(core_axis_name="c", subcore_axis_name="s")
    fn = pl.kernel(
        _sc_body,
        out_type=jax.ShapeDtypeStruct((BH, COUT, H, W), jnp.float32),
        mesh=mesh,
        compiler_params=pltpu.CompilerParams(needs_layout_passes=False),
        scratch_types=[
            pltpu.VMEM((TROWS_PER_PLANE * 128,), jnp.float32),
            pltpu.VMEM((TROWS_PER_PLANE * 128,), jnp.float32),
            pltpu.VMEM((OCH, W), jnp.int32),
            pltpu.VMEM((OCH, W), jnp.int32),
            pltpu.VMEM((OCH, W), jnp.float32),
            pltpu.VMEM((OCH, W), jnp.float32),
            pltpu.SemaphoreType.DMA,
            pltpu.SemaphoreType.DMA,
            pltpu.SemaphoreType.DMA,
            pltpu.SemaphoreType.DMA,
            pltpu.SemaphoreType.DMA,
            pltpu.SemaphoreType.DMA,
        ],
    )
    return fn(tbl, idx)


@jax.jit
def kernel(x):
    x_p = jnp.transpose(x, (0, 3, 1, 2))
    halves = []
    for base in (0, BH):
        tbl, idx = _tc_stage(x_p, base)
        halves.append(_sc_stage(tbl.reshape(-1), idx))
    out_p = jnp.concatenate(halves, axis=0)
    return jnp.transpose(out_p, (0, 2, 3, 1))


# speedup vs baseline: 1.0250x; 1.0250x over previous
import jax
import jax.numpy as jnp
from jax import lax
from jax.experimental import pallas as pl
from jax.experimental.pallas import tpu as pltpu
from jax.experimental.pallas import tpu_sc as plsc

B = 64
H = 224
W = 224
CIN = 5
COUT = 3

TROWS_PER_PLANE = 2 * H
TROWS_PER_IMG = COUT * TROWS_PER_PLANE
TBL_ROWS = B * TROWS_PER_IMG
RIGHT_OFF = 28576

NC = 2
NS = 16
NW = NC * NS
BH = B // 2
OCH = 16
NCHUNK = H // OCH


def _tc_body(x_ref, tbl_ref, idx_ref):
    v = x_ref[0]
    for c in range(COUT):
        p = v[c]
        s = (p[0:222, 0:222] + p[0:222, 2:224]
             + p[2:224, 0:222] + p[2:224, 2:224]) * 0.25
        r = c * TROWS_PER_PLANE
        tbl_ref[r:r + TROWS_PER_PLANE, :] = jnp.zeros(
            (TROWS_PER_PLANE, 128), jnp.float32)
        tbl_ref[r + 1:r + 223, 1:128] = s[:, 0:127]
        tbl_ref[r + H + 1:r + H + 223, 0:127] = s[:, 95:222]

    ii = lax.broadcasted_iota(jnp.int32, (H, W), 0).astype(jnp.float32)
    jj = lax.broadcasted_iota(jnp.int32, (H, W), 1).astype(jnp.float32)
    yi = jnp.clip(jnp.mod(ii + v[4], 224.0).astype(jnp.int32), 0, 223)
    xi = jnp.clip(jnp.mod(jj + v[3], 224.0).astype(jnp.int32), 0, 223)
    idx_ref[0] = yi * 128 + xi + jnp.where(xi >= 128, RIGHT_OFF, 0)


def _tc_stage(x_p, base):
    return pl.pallas_call(
        _tc_body,
        grid=(BH,),
        in_specs=[pl.BlockSpec((1, CIN, H, W),
                               lambda b: (b + base, 0, 0, 0))],
        out_specs=[
            pl.BlockSpec((TROWS_PER_IMG, 128), lambda b: (b, 0)),
            pl.BlockSpec((1, H, W), lambda b: (b, 0, 0)),
        ],
        out_shape=[
            jax.ShapeDtypeStruct((BH * TROWS_PER_IMG, 128), jnp.float32),
            jax.ShapeDtypeStruct((BH, H, W), jnp.int32),
        ],
    )(x_p)


def _sc_body(tbl_hbm, idx_hbm, out_hbm,
             tl0, tl1, ib0, ib1, ob0, ob1,
             st0, st1, si0, si1, so0, so1):
    wid = lax.axis_index("s") * NC + lax.axis_index("c")
    b = wid
    tbufs, ibufs, obufs = (tl0, tl1), (ib0, ib1), (ob0, ob1)
    stems, siems, soems = (st0, st1), (si0, si1), (so0, so1)

    def tbl_src(c):
        r0 = (b * TROWS_PER_IMG + c * TROWS_PER_PLANE) * 128
        return tbl_hbm.at[pl.ds(r0, TROWS_PER_PLANE * 128)]

    def idx_src(g):
        return idx_hbm.at[b, pl.ds(g * OCH, OCH)]

    pltpu.async_copy(tbl_src(0), tbufs[0], stems[0])
    pltpu.async_copy(idx_src(0), ibufs[0], siems[0])
    pend_out = [None, None]

    for c in range(COUT):
        tb = tbufs[c % 2]
        pltpu.make_async_copy(tbl_src(c), tb, stems[c % 2]).wait()
        if c + 1 < COUT:
            pltpu.async_copy(tbl_src(c + 1), tbufs[(c + 1) % 2],
                             stems[(c + 1) % 2])
        for gg in range(NCHUNK):
            q = c * NCHUNK + gg
            p = q % 2
            ib = ibufs[p]
            pltpu.make_async_copy(idx_src(gg), ib, siems[p]).wait()
            if q + 1 < COUT * NCHUNK:
                pltpu.async_copy(idx_src((gg + 1) % NCHUNK),
                                 ibufs[1 - p], siems[1 - p])
            if pend_out[p] is not None:
                ob_, od_ = pend_out[p]
                pltpu.make_async_copy(ob_, od_, soems[p]).wait()
            ob = obufs[p]

            def fill(rr, carry, _ib=ib, _ob=ob, _tb=tb):
                for t in range(W // 16):
                    pv = _ib[rr, pl.ds(t * 16, 16)]
                    _ob[rr, pl.ds(t * 16, 16)] = plsc.load_gather(_tb, [pv])
                return carry

            lax.fori_loop(0, OCH, fill, 0)
            dst = out_hbm.at[b, c, pl.ds(gg * OCH, OCH)]
            pltpu.async_copy(ob, dst, soems[p])
            pend_out[p] = (ob, dst)
    for p in range(2):
        if pend_out[p] is not None:
            ob_, od_ = pend_out[p]
            pltpu.make_async_copy(ob_, od_, soems[p]).wait()


def _sc_stage(tbl, idx):
    mesh = plsc.VectorSubcoreMesh(core_axis_name="c", subcore_axis_name="s")
    fn = pl.kernel(
        _sc_body,
        out_type=jax.ShapeDtypeStruct((BH, COUT, H, W), jnp.float32),
        mesh=mesh,
        compiler_params=pltpu.CompilerParams(needs_layout_passes=False),
        scratch_types=[
            pltpu.VMEM((TROWS_PER_PLANE * 128,), jnp.float32),
            pltpu.VMEM((TROWS_PER_PLANE * 128,), jnp.float32),
            pltpu.VMEM((OCH, W), jnp.int32),
            pltpu.VMEM((OCH, W), jnp.int32),
            pltpu.VMEM((OCH, W), jnp.float32),
            pltpu.VMEM((OCH, W), jnp.float32),
            pltpu.SemaphoreType.DMA,
            pltpu.SemaphoreType.DMA,
            pltpu.SemaphoreType.DMA,
            pltpu.SemaphoreType.DMA,
            pltpu.SemaphoreType.DMA,
            pltpu.SemaphoreType.DMA,
        ],
    )
    return fn(tbl, idx)


@jax.jit
def kernel(x):
    x_p = jnp.transpose(x, (0, 3, 1, 2))
    halves = []
    for base in (0, BH):
        tbl, idx = _tc_stage(x_p, base)
        halves.append(_sc_stage(tbl.reshape(-1), idx))
    out_p = jnp.zeros((B, COUT, H, W), jnp.float32)
    out_p = lax.dynamic_update_slice(out_p, halves[0], (0, 0, 0, 0))
    out_p = lax.dynamic_update_slice(out_p, halves[1], (BH, 0, 0, 0))
    return jnp.transpose(out_p, (0, 2, 3, 1))
